# baseline (speedup 1.0000x reference)
import functools
import math

import jax
import jax.numpy as jnp
from jax import lax
from jax.experimental import pallas as pl
from jax.experimental.pallas import tpu as pltpu
from jax.experimental.pallas import tpu_sc as plsc

N = 4096
D_SPAN = 3092
KP = 3200
DP = 4096
K_TOP = 50
K_PAD = 64
SENT = float(-1e30)
SENT2 = float(-3e38)

_BM1 = 256
_BN1 = 1024


def _t1_body(span_ref, w_ref, b_ref, out_ref):
    acc = lax.dot_general(
        span_ref[...].astype(jnp.bfloat16), w_ref[...].astype(jnp.bfloat16),
        dimension_numbers=(((1,), (1,)), ((), ())),
        preferred_element_type=jnp.float32,
    )
    out_ref[...] = acc + b_ref[...]


def _t1(span_p, w_p, b_p):
    grid = (DP // _BN1, N // _BM1)
    return pl.pallas_call(
        _t1_body,
        grid=grid,
        in_specs=[
            pl.BlockSpec((_BM1, KP), lambda j, i: (i, 0)),
            pl.BlockSpec((_BN1, KP), lambda j, i: (j, 0)),
            pl.BlockSpec((1, _BN1), lambda j, i: (0, j)),
        ],
        out_specs=pl.BlockSpec((_BM1, _BN1), lambda j, i: (i, j)),
        out_shape=jax.ShapeDtypeStruct((N, DP), jnp.float32),
    )(span_p, w_p, b_p)


_BM2 = 256
_BN2 = 1024
_INV_LOG2 = 1.0 / math.log(2.0)


def _t2_body(src_ref, span_ref, msr_ref, msc_ref, bs_ref, out_ref):
    j = pl.program_id(0)
    i = pl.program_id(1)
    row0 = i * _BM2
    col0 = j * _BN2

    rows = row0 + lax.broadcasted_iota(jnp.int32, (_BM2, _BN2), 0)
    cols = col0 + lax.broadcasted_iota(jnp.int32, (_BM2, _BN2), 1)
    off = rows - cols
    valid = off >= 1

    @pl.when(row0 + _BM2 - 1 >= col0 + 1)
    def _compute():
        acc = lax.dot_general(
            src_ref[...].astype(jnp.bfloat16),
            span_ref[...].astype(jnp.bfloat16),
            dimension_numbers=(((1,), (1,)), ((), ())),
            preferred_element_type=jnp.float32,
        )
        d = jnp.maximum(off.astype(jnp.float32), 1.0)
        lg = jnp.floor(jnp.log(d) * _INV_LOG2).astype(jnp.int32) + 3
        bucket = jnp.clip(jnp.where(off <= 4, off, lg), 0, 9)
        bval = jnp.zeros((_BM2, _BN2), jnp.float32)
        for t in range(10):
            bval = jnp.where(bucket == t, bs_ref[0, t], bval)
        s = (msr_ref[...] + msc_ref[...]) + acc
        s = s + bval
        out_ref[...] = jnp.where(valid, s, SENT)

    @pl.when(row0 + _BM2 - 1 < col0 + 1)
    def _masked():
        out_ref[...] = jnp.full((_BM2, _BN2), SENT, jnp.float32)


def _t2(source_p, span_p, ms_row, ms_col, bs_pad):
    grid = (N // _BN2, N // _BM2)
    return pl.pallas_call(
        _t2_body,
        grid=grid,
        in_specs=[
            pl.BlockSpec((_BM2, KP), lambda j, i: (i, 0)),
            pl.BlockSpec((_BN2, KP), lambda j, i: (j, 0)),
            pl.BlockSpec((_BM2, 1), lambda j, i: (i, 0)),
            pl.BlockSpec((1, _BN2), lambda j, i: (0, j)),
            pl.BlockSpec((1, 128), lambda j, i: (0, 0)),
        ],
        out_specs=pl.BlockSpec((_BM2, _BN2), lambda j, i: (i, j)),
        out_shape=jax.ShapeDtypeStruct((N, N), jnp.float32),
    )(source_p, span_p, ms_row, ms_col, bs_pad)


_NC = 2
_NS = 16
_NW = _NC * _NS
_ROWS_PER_W = N // _NW
_NCHUNK = N // 16


def _sc_topk_body(scores_hbm, idx_hbm, val_hbm, off_hbm, msk_hbm,
                  rbuf_a, rbuf_b, rbuf_c, rbuf_d,
                  mbuf_x, m2buf_x, mbuf_y, m2buf_y, obi, obs, obo, obm,
                  sem_a, sem_b, sem_c, sem_d):
    cid = lax.axis_index("c")
    sid = lax.axis_index("s")
    wid = (sid * _NC + cid).astype(jnp.int32)
    lanes = lax.iota(jnp.int32, 16)
    mask0 = lanes == 0
    neg_inf = jnp.float32(-jnp.inf)

    def build(i, rbuf, mbuf, m2buf):
        nvalid = jnp.maximum(i, K_TOP)
        nchunks = (nvalid + 15) // 16
        mcs = (nchunks + 15) // 16

        @pl.loop(0, mcs)
        def _chunkmax(mc):
            starts = mc * 256 + lanes * 16
            mv = plsc.load_gather(rbuf, [starts])
            for l in range(1, 16):
                mv = jnp.maximum(mv, plsc.load_gather(rbuf, [starts + l]))
            plsc.store_scatter(mbuf, [mc * 16 + lanes], mv)
            plsc.store_scatter(m2buf, [jnp.full((16,), mc)],
                               jnp.full((16,), jnp.max(mv)), mask=mask0)

        m2buf[...] = jnp.where(lanes < mcs, m2buf[...], SENT2)

    def pick1(rbuf, mbuf, m2buf, iv, slotv, kk):
        m2v = m2buf[...]
        gs = jnp.max(m2v)
        gsv = jnp.full((16,), gs)
        gv = plsc.all_reduce_ffs(m2v == gsv)
        mv = plsc.load_gather(mbuf, [gv * 16 + lanes])
        cv = plsc.all_reduce_ffs(mv == gsv)
        base = (gv * 16 + cv) * 16
        vec = plsc.load_gather(rbuf, [base + lanes])
        lv = plsc.all_reduce_ffs(vec == gsv)
        colv = base + lv

        is_pad = gsv <= SENT
        plsc.store_scatter(obi, [slotv, kk], colv, mask=mask0)
        plsc.store_scatter(obs, [slotv, kk],
                           jnp.where(is_pad, neg_inf, gsv), mask=mask0)
        plsc.store_scatter(obo, [slotv, kk], iv - colv, mask=mask0)
        plsc.store_scatter(obm, [slotv, kk],
                           jnp.where(is_pad, 0, 1).astype(jnp.int32),
                           mask=mask0)

        vec2 = jnp.where(lanes == lv, SENT2, vec)
        plsc.store_scatter(rbuf, [base + lanes], vec2)
        ncm = jnp.full((16,), jnp.max(vec2))
        plsc.store_scatter(mbuf, [gv * 16 + cv], ncm, mask=mask0)
        mv2 = jnp.where(lanes == cv, ncm, mv)
        plsc.store_scatter(m2buf, [gv], jnp.full((16,), jnp.max(mv2)),
                           mask=mask0)

    def process2(iA, rbA, slotA, iB, rbB, slotB):
        build(iA, rbA, mbuf_x, m2buf_x)
        build(iB, rbB, mbuf_y, m2buf_y)
        ivA = jnp.full((16,), iA)
        svA = jnp.full((16,), slotA)
        ivB = jnp.full((16,), iB)
        svB = jnp.full((16,), slotB)

        @pl.loop(0, K_TOP)
        def _pick(k):
            kk = jnp.full((16,), k)
            pick1(rbA, mbuf_x, m2buf_x, ivA, svA, kk)
            pick1(rbB, mbuf_y, m2buf_y, ivB, svB, kk)

    def issue(r, rbuf, sem):
        pltpu.async_copy(scores_hbm.at[r * _NW + wid], rbuf, sem)

    def wait(rbuf, sem):
        pltpu.make_async_copy(scores_hbm.at[0], rbuf, sem).wait()

    def flush(b8):
        b = pl.multiple_of(b8, 8)
        pltpu.sync_copy(obi, idx_hbm.at[wid, pl.ds(b, 8)])
        pltpu.sync_copy(obs, val_hbm.at[wid, pl.ds(b, 8)])
        pltpu.sync_copy(obo, off_hbm.at[wid, pl.ds(b, 8)])
        pltpu.sync_copy(obm, msk_hbm.at[wid, pl.ds(b, 8)])

    issue(0, rbuf_a, sem_a)
    issue(1, rbuf_b, sem_b)
    issue(2, rbuf_c, sem_c)
    issue(3, rbuf_d, sem_d)

    @pl.loop(0, _ROWS_PER_W // 4)
    def _u(u):
        base = 4 * u
        wait(rbuf_a, sem_a)
        wait(rbuf_b, sem_b)
        process2(base * _NW + wid, rbuf_a, base % 8,
                 (base + 1) * _NW + wid, rbuf_b, (base + 1) % 8)

        @pl.when(base + 4 < _ROWS_PER_W)
        def _issue_ab():
            issue(base + 4, rbuf_a, sem_a)
            issue(base + 5, rbuf_b, sem_b)

        wait(rbuf_c, sem_c)
        wait(rbuf_d, sem_d)
        process2((base + 2) * _NW + wid, rbuf_c, (base + 2) % 8,
                 (base + 3) * _NW + wid, rbuf_d, (base + 3) % 8)

        @pl.when(base + 6 < _ROWS_PER_W)
        def _issue_cd():
            issue(base + 6, rbuf_c, sem_c)
            issue(base + 7, rbuf_d, sem_d)

        @pl.when(u % 2 == 1)
        def _flush():
            flush(base - 4)


@functools.cache
def _get_sc_topk():
    return _sc_topk_partial()(_sc_topk_body)


def _sc_topk_partial():
    return functools.partial(
        pl.kernel,
        out_type=(
            jax.ShapeDtypeStruct((_NW, _ROWS_PER_W, K_PAD), jnp.int32),
            jax.ShapeDtypeStruct((_NW, _ROWS_PER_W, K_PAD), jnp.float32),
            jax.ShapeDtypeStruct((_NW, _ROWS_PER_W, K_PAD), jnp.int32),
            jax.ShapeDtypeStruct((_NW, _ROWS_PER_W, K_PAD), jnp.int32),
        ),
        mesh=plsc.VectorSubcoreMesh(
            core_axis_name="c", subcore_axis_name="s",
            num_cores=_NC, num_subcores=_NS),
        compiler_params=pltpu.CompilerParams(needs_layout_passes=False),
        scratch_types=[
            pltpu.VMEM((N,), jnp.float32),
            pltpu.VMEM((N,), jnp.float32),
            pltpu.VMEM((N,), jnp.float32),
            pltpu.VMEM((N,), jnp.float32),
            pltpu.VMEM((_NCHUNK,), jnp.float32),
            pltpu.VMEM((16,), jnp.float32),
            pltpu.VMEM((_NCHUNK,), jnp.float32),
            pltpu.VMEM((16,), jnp.float32),
            pltpu.VMEM((8, K_PAD), jnp.int32),
            pltpu.VMEM((8, K_PAD), jnp.float32),
            pltpu.VMEM((8, K_PAD), jnp.int32),
            pltpu.VMEM((8, K_PAD), jnp.int32),
            pltpu.SemaphoreType.DMA,
            pltpu.SemaphoreType.DMA,
            pltpu.SemaphoreType.DMA,
            pltpu.SemaphoreType.DMA,
        ],
    )


def kernel(span_emb, mention_scores, num_top_antecedents, W_fast, b_fast,
           emb_fast_distance, W_dist, b_dist):
    del num_top_antecedents
    span_p = jnp.pad(span_emb, ((0, 0), (0, KP - D_SPAN)))
    w_p = jnp.pad(W_fast, ((0, DP - D_SPAN), (0, KP - D_SPAN)))
    b_p = jnp.pad(b_fast, (0, DP - D_SPAN)).reshape(1, DP)

    source_p = _t1(span_p, w_p, b_p)

    bs10 = (emb_fast_distance @ W_dist.T + b_dist)[:, 0]
    bs_pad = jnp.zeros((1, 128), jnp.float32).at[0, :10].set(bs10)
    ms_row = mention_scores.reshape(N, 1)
    ms_col = mention_scores.reshape(1, N)

    scores = _t2(source_p, span_p, ms_row, ms_col, bs_pad)

    idx, val, off, msk = _get_sc_topk()(scores)

    def _unshard(x):
        return x.transpose(1, 0, 2).reshape(N, K_PAD)[:, :K_TOP]

    return (_unshard(idx), _unshard(msk).astype(bool),
            _unshard(val), _unshard(off))


# speedup vs baseline: 1.0006x; 1.0006x over previous
import functools
import math

import jax
import jax.numpy as jnp
from jax import lax
from jax.experimental import pallas as pl
from jax.experimental.pallas import tpu as pltpu
from jax.experimental.pallas import tpu_sc as plsc

N = 4096
D_SPAN = 3092
KP = 3200
DP = 4096
K_TOP = 50
K_PAD = 64
SENT = float(-1e30)
SENT2 = float(-3e38)

_BM1 = 256
_BN1 = 1024


def _t1_body(span_ref, w_ref, b_ref, out_ref):
    acc = lax.dot_general(
        span_ref[...].astype(jnp.bfloat16), w_ref[...].astype(jnp.bfloat16),
        dimension_numbers=(((1,), (1,)), ((), ())),
        preferred_element_type=jnp.float32,
    )
    out_ref[...] = acc + b_ref[...]


def _t1(span_p, w_p, b_p):
    grid = (DP // _BN1, N // _BM1)
    return pl.pallas_call(
        _t1_body,
        grid=grid,
        in_specs=[
            pl.BlockSpec((_BM1, KP), lambda j, i: (i, 0)),
            pl.BlockSpec((_BN1, KP), lambda j, i: (j, 0)),
            pl.BlockSpec((1, _BN1), lambda j, i: (0, j)),
        ],
        out_specs=pl.BlockSpec((_BM1, _BN1), lambda j, i: (i, j)),
        out_shape=jax.ShapeDtypeStruct((N, DP), jnp.float32),
    )(span_p, w_p, b_p)


_BM2 = 256
_BN2 = 1024
_INV_LOG2 = 1.0 / math.log(2.0)


def _t2_body(src_ref, span_ref, msr_ref, msc_ref, bs_ref, out_ref):
    j = pl.program_id(0)
    i = pl.program_id(1)
    row0 = i * _BM2
    col0 = j * _BN2

    rows = row0 + lax.broadcasted_iota(jnp.int32, (_BM2, _BN2), 0)
    cols = col0 + lax.broadcasted_iota(jnp.int32, (_BM2, _BN2), 1)
    off = rows - cols
    valid = off >= 1

    @pl.when(row0 + _BM2 - 1 >= col0 + 1)
    def _compute():
        acc = lax.dot_general(
            src_ref[...].astype(jnp.bfloat16),
            span_ref[...].astype(jnp.bfloat16),
            dimension_numbers=(((1,), (1,)), ((), ())),
            preferred_element_type=jnp.float32,
        )
        d = jnp.maximum(off.astype(jnp.float32), 1.0)
        lg = jnp.floor(jnp.log(d) * _INV_LOG2).astype(jnp.int32) + 3
        bucket = jnp.clip(jnp.where(off <= 4, off, lg), 0, 9)
        bval = jnp.zeros((_BM2, _BN2), jnp.float32)
        for t in range(10):
            bval = jnp.where(bucket == t, bs_ref[0, t], bval)
        s = (msr_ref[...] + msc_ref[...]) + acc
        s = s + bval
        out_ref[...] = jnp.where(valid, s, SENT)

    @pl.when(row0 + _BM2 - 1 < col0 + 1)
    def _masked():
        out_ref[...] = jnp.full((_BM2, _BN2), SENT, jnp.float32)


def _t2(source_p, span_p, ms_row, ms_col, bs_pad):
    grid = (N // _BN2, N // _BM2)
    return pl.pallas_call(
        _t2_body,
        grid=grid,
        in_specs=[
            pl.BlockSpec((_BM2, KP), lambda j, i: (i, 0)),
            pl.BlockSpec((_BN2, KP), lambda j, i: (j, 0)),
            pl.BlockSpec((_BM2, 1), lambda j, i: (i, 0)),
            pl.BlockSpec((1, _BN2), lambda j, i: (0, j)),
            pl.BlockSpec((1, 128), lambda j, i: (0, 0)),
        ],
        out_specs=pl.BlockSpec((_BM2, _BN2), lambda j, i: (i, j)),
        out_shape=jax.ShapeDtypeStruct((N, N), jnp.float32),
    )(source_p, span_p, ms_row, ms_col, bs_pad)


_NC = 2
_NS = 16
_NW = _NC * _NS
_ROWS_PER_W = N // _NW
_NCHUNK = N // 16


def _sc_topk_body(scores_hbm, idx_hbm, val_hbm, off_hbm, msk_hbm,
                  rbuf_a, rbuf_b, mbuf, m2buf, obi, obs, obo, obm,
                  sem_a, sem_b):
    cid = lax.axis_index("c")
    sid = lax.axis_index("s")
    wid = (sid * _NC + cid).astype(jnp.int32)
    lanes = lax.iota(jnp.int32, 16)
    mask0 = lanes == 0
    neg_inf = jnp.float32(-jnp.inf)

    def process(i, rbuf, slot):
        nvalid = jnp.maximum(i, K_TOP)
        nchunks = (nvalid + 15) // 16
        mcs = (nchunks + 15) // 16

        @pl.loop(0, mcs)
        def _chunkmax(mc):
            starts = mc * 256 + lanes * 16
            mv = plsc.load_gather(rbuf, [starts])
            for l in range(1, 16):
                mv = jnp.maximum(mv, plsc.load_gather(rbuf, [starts + l]))
            plsc.store_scatter(mbuf, [mc * 16 + lanes], mv)
            plsc.store_scatter(m2buf, [jnp.full((16,), mc)],
                               jnp.full((16,), jnp.max(mv)), mask=mask0)

        m2buf[...] = jnp.where(lanes < mcs, m2buf[...], SENT2)

        iv = jnp.full((16,), i)
        slotv = jnp.full((16,), slot)

        @pl.loop(0, K_TOP)
        def _pick(k):
            m2v = m2buf[...]
            gs = jnp.max(m2v)
            gsv = jnp.full((16,), gs)
            gv = plsc.all_reduce_ffs(m2v == gsv)
            mv = plsc.load_gather(mbuf, [gv * 16 + lanes])
            cv = plsc.all_reduce_ffs(mv == gsv)
            base = (gv * 16 + cv) * 16
            vec = plsc.load_gather(rbuf, [base + lanes])
            lv = plsc.all_reduce_ffs(vec == gsv)
            colv = base + lv

            kk = jnp.full((16,), k)
            is_pad = gsv <= SENT
            plsc.store_scatter(obi, [slotv, kk], colv, mask=mask0)
            plsc.store_scatter(obs, [slotv, kk],
                               jnp.where(is_pad, neg_inf, gsv), mask=mask0)
            plsc.store_scatter(obo, [slotv, kk], iv - colv, mask=mask0)
            plsc.store_scatter(obm, [slotv, kk],
                               jnp.where(is_pad, 0, 1).astype(jnp.int32),
                               mask=mask0)

            vec2 = jnp.where(lanes == lv, SENT2, vec)
            plsc.store_scatter(rbuf, [base + lanes], vec2)
            ncm = jnp.full((16,), jnp.max(vec2))
            plsc.store_scatter(mbuf, [gv * 16 + cv], ncm, mask=mask0)
            mv2 = jnp.where(lanes == cv, ncm, mv)
            plsc.store_scatter(m2buf, [gv], jnp.full((16,), jnp.max(mv2)),
                               mask=mask0)

    pltpu.async_copy(scores_hbm.at[wid], rbuf_a, sem_a)

    @pl.loop(0, _ROWS_PER_W // 2)
    def _t(t):
        r0 = 2 * t
        r1 = r0 + 1
        pltpu.async_copy(scores_hbm.at[r1 * _NW + wid], rbuf_b, sem_b)
        pltpu.make_async_copy(scores_hbm.at[0], rbuf_a, sem_a).wait()
        process(r0 * _NW + wid, rbuf_a, r0 % 8)

        @pl.when(r0 + 2 < _ROWS_PER_W)
        def _issue_a():
            pltpu.async_copy(scores_hbm.at[(r0 + 2) * _NW + wid], rbuf_a,
                             sem_a)

        pltpu.make_async_copy(scores_hbm.at[0], rbuf_b, sem_b).wait()
        process(r1 * _NW + wid, rbuf_b, r1 % 8)

        @pl.when(t % 4 == 3)
        def _flush():
            b = pl.multiple_of(r1 - 7, 8)
            pltpu.sync_copy(obi, idx_hbm.at[wid, pl.ds(b, 8)])
            pltpu.sync_copy(obs, val_hbm.at[wid, pl.ds(b, 8)])
            pltpu.sync_copy(obo, off_hbm.at[wid, pl.ds(b, 8)])
            pltpu.sync_copy(obm, msk_hbm.at[wid, pl.ds(b, 8)])


@functools.cache
def _get_sc_topk():
    return _sc_topk_partial()(_sc_topk_body)


def _sc_topk_partial():
    return functools.partial(
        pl.kernel,
        out_type=(
            jax.ShapeDtypeStruct((_NW, _ROWS_PER_W, K_PAD), jnp.int32),
            jax.ShapeDtypeStruct((_NW, _ROWS_PER_W, K_PAD), jnp.float32),
            jax.ShapeDtypeStruct((_NW, _ROWS_PER_W, K_PAD), jnp.int32),
            jax.ShapeDtypeStruct((_NW, _ROWS_PER_W, K_PAD), jnp.int32),
        ),
        mesh=plsc.VectorSubcoreMesh(
            core_axis_name="c", subcore_axis_name="s",
            num_cores=_NC, num_subcores=_NS),
        compiler_params=pltpu.CompilerParams(needs_layout_passes=False),
        scratch_types=[
            pltpu.VMEM((N,), jnp.float32),
            pltpu.VMEM((N,), jnp.float32),
            pltpu.VMEM((_NCHUNK,), jnp.float32),
            pltpu.VMEM((16,), jnp.float32),
            pltpu.VMEM((8, K_PAD), jnp.int32),
            pltpu.VMEM((8, K_PAD), jnp.float32),
            pltpu.VMEM((8, K_PAD), jnp.int32),
            pltpu.VMEM((8, K_PAD), jnp.int32),
            pltpu.SemaphoreType.DMA,
            pltpu.SemaphoreType.DMA,
        ],
    )


def kernel(span_emb, mention_scores, num_top_antecedents, W_fast, b_fast,
           emb_fast_distance, W_dist, b_dist):
    del num_top_antecedents
    span_p = jnp.pad(span_emb, ((0, 0), (0, KP - D_SPAN)))
    w_p = jnp.pad(W_fast, ((0, DP - D_SPAN), (0, KP - D_SPAN)))
    b_p = jnp.pad(b_fast, (0, DP - D_SPAN)).reshape(1, DP)

    source_p = _t1(span_p, w_p, b_p)

    bs10 = (emb_fast_distance @ W_dist.T + b_dist)[:, 0]
    bs_pad = jnp.zeros((1, 128), jnp.float32).at[0, :10].set(bs10)
    ms_row = mention_scores.reshape(N, 1)
    ms_col = mention_scores.reshape(1, N)

    scores = _t2(source_p, span_p, ms_row, ms_col, bs_pad)

    idx, val, off, msk = _get_sc_topk()(scores)

    def _unshard(x):
        return x.transpose(1, 0, 2).reshape(N, K_PAD)[:, :K_TOP]

    return (_unshard(idx), _unshard(msk).astype(bool),
            _unshard(val), _unshard(off))


# speedup vs baseline: 1.0206x; 1.0200x over previous
import functools
import math

import jax
import jax.numpy as jnp
from jax import lax
from jax.experimental import pallas as pl
from jax.experimental.pallas import tpu as pltpu
from jax.experimental.pallas import tpu_sc as plsc

N = 4096
D_SPAN = 3092
KP = 3200
DP = 4096
K_TOP = 50
K_PAD = 64
SENT = float(-1e30)
SENT2 = float(-3e38)

_BM1 = 256
_BN1 = 2048


def _t1_body(span_ref, w_ref, b_ref, out_ref):
    acc = lax.dot_general(
        span_ref[...].astype(jnp.bfloat16), w_ref[...].astype(jnp.bfloat16),
        dimension_numbers=(((1,), (1,)), ((), ())),
        preferred_element_type=jnp.float32,
    )
    out_ref[...] = acc + b_ref[...]


def _t1(span_p, w_p, b_p):
    grid = (DP // _BN1, N // _BM1)
    return pl.pallas_call(
        _t1_body,
        grid=grid,
        in_specs=[
            pl.BlockSpec((_BM1, KP), lambda j, i: (i, 0)),
            pl.BlockSpec((_BN1, KP), lambda j, i: (j, 0)),
            pl.BlockSpec((1, _BN1), lambda j, i: (0, j)),
        ],
        out_specs=pl.BlockSpec((_BM1, _BN1), lambda j, i: (i, j)),
        out_shape=jax.ShapeDtypeStruct((N, DP), jnp.float32),
        compiler_params=pltpu.CompilerParams(
            vmem_limit_bytes=120 * 1024 * 1024),
    )(span_p, w_p, b_p)


_BM2 = 256
_BN2 = 1024
_INV_LOG2 = 1.0 / math.log(2.0)


def _t2_body(src_ref, span_ref, msr_ref, msc_ref, bs_ref, out_ref):
    j = pl.program_id(0)
    i = pl.program_id(1)
    row0 = i * _BM2
    col0 = j * _BN2

    rows = row0 + lax.broadcasted_iota(jnp.int32, (_BM2, _BN2), 0)
    cols = col0 + lax.broadcasted_iota(jnp.int32, (_BM2, _BN2), 1)
    off = rows - cols
    valid = off >= 1

    @pl.when(row0 + _BM2 - 1 >= col0 + 1)
    def _compute():
        acc = lax.dot_general(
            src_ref[...].astype(jnp.bfloat16),
            span_ref[...].astype(jnp.bfloat16),
            dimension_numbers=(((1,), (1,)), ((), ())),
            preferred_element_type=jnp.float32,
        )
        d = jnp.maximum(off.astype(jnp.float32), 1.0)
        lg = jnp.floor(jnp.log(d) * _INV_LOG2).astype(jnp.int32) + 3
        bucket = jnp.clip(jnp.where(off <= 4, off, lg), 0, 9)
        bval = jnp.zeros((_BM2, _BN2), jnp.float32)
        for t in range(10):
            bval = jnp.where(bucket == t, bs_ref[0, t], bval)
        s = (msr_ref[...] + msc_ref[...]) + acc
        s = s + bval
        out_ref[...] = jnp.where(valid, s, SENT)

    @pl.when(row0 + _BM2 - 1 < col0 + 1)
    def _masked():
        out_ref[...] = jnp.full((_BM2, _BN2), SENT, jnp.float32)


def _t2(source_p, span_p, ms_row, ms_col, bs_pad):
    grid = (N // _BN2, N // _BM2)
    return pl.pallas_call(
        _t2_body,
        grid=grid,
        in_specs=[
            pl.BlockSpec((_BM2, KP), lambda j, i: (i, 0)),
            pl.BlockSpec((_BN2, KP), lambda j, i: (j, 0)),
            pl.BlockSpec((_BM2, 1), lambda j, i: (i, 0)),
            pl.BlockSpec((1, _BN2), lambda j, i: (0, j)),
            pl.BlockSpec((1, 128), lambda j, i: (0, 0)),
        ],
        out_specs=pl.BlockSpec((_BM2, _BN2), lambda j, i: (i, j)),
        out_shape=jax.ShapeDtypeStruct((N, N), jnp.float32),
        compiler_params=pltpu.CompilerParams(
            vmem_limit_bytes=120 * 1024 * 1024),
    )(source_p, span_p, ms_row, ms_col, bs_pad)


_NC = 2
_NS = 16
_NW = _NC * _NS
_ROWS_PER_W = N // _NW
_NCHUNK = N // 16


def _sc_topk_body(scores_hbm, idx_hbm, val_hbm, off_hbm, msk_hbm,
                  rbuf_a, rbuf_b, mbuf, m2buf, obi, obs, obo, obm,
                  sem_a, sem_b):
    cid = lax.axis_index("c")
    sid = lax.axis_index("s")
    wid = (sid * _NC + cid).astype(jnp.int32)
    lanes = lax.iota(jnp.int32, 16)
    mask0 = lanes == 0
    neg_inf = jnp.float32(-jnp.inf)

    def process(i, rbuf, slot):
        nvalid = jnp.maximum(i, K_TOP)
        nchunks = (nvalid + 15) // 16
        mcs = (nchunks + 15) // 16

        @pl.loop(0, mcs)
        def _chunkmax(mc):
            starts = mc * 256 + lanes * 16
            mv = plsc.load_gather(rbuf, [starts])
            for l in range(1, 16):
                mv = jnp.maximum(mv, plsc.load_gather(rbuf, [starts + l]))
            plsc.store_scatter(mbuf, [mc * 16 + lanes], mv)
            plsc.store_scatter(m2buf, [jnp.full((16,), mc)],
                               jnp.full((16,), jnp.max(mv)), mask=mask0)

        m2buf[...] = jnp.where(lanes < mcs, m2buf[...], SENT2)

        iv = jnp.full((16,), i)
        slotv = jnp.full((16,), slot)

        @pl.loop(0, K_TOP)
        def _pick(k):
            m2v = m2buf[...]
            gs = jnp.max(m2v)
            gsv = jnp.full((16,), gs)
            gv = plsc.all_reduce_ffs(m2v == gsv)
            mv = plsc.load_gather(mbuf, [gv * 16 + lanes])
            cv = plsc.all_reduce_ffs(mv == gsv)
            base = (gv * 16 + cv) * 16
            vec = plsc.load_gather(rbuf, [base + lanes])
            lv = plsc.all_reduce_ffs(vec == gsv)
            colv = base + lv

            kk = jnp.full((16,), k)
            is_pad = gsv <= SENT
            plsc.store_scatter(obi, [slotv, kk], colv, mask=mask0)
            plsc.store_scatter(obs, [slotv, kk],
                               jnp.where(is_pad, neg_inf, gsv), mask=mask0)
            plsc.store_scatter(obo, [slotv, kk], iv - colv, mask=mask0)
            plsc.store_scatter(obm, [slotv, kk],
                               jnp.where(is_pad, 0, 1).astype(jnp.int32),
                               mask=mask0)

            vec2 = jnp.where(lanes == lv, SENT2, vec)
            plsc.store_scatter(rbuf, [base + lanes], vec2)
            ncm = jnp.full((16,), jnp.max(vec2))
            plsc.store_scatter(mbuf, [gv * 16 + cv], ncm, mask=mask0)
            mv2 = jnp.where(lanes == cv, ncm, mv)
            plsc.store_scatter(m2buf, [gv], jnp.full((16,), jnp.max(mv2)),
                               mask=mask0)

    pltpu.async_copy(scores_hbm.at[wid], rbuf_a, sem_a)

    @pl.loop(0, _ROWS_PER_W // 2)
    def _t(t):
        r0 = 2 * t
        r1 = r0 + 1
        pltpu.async_copy(scores_hbm.at[r1 * _NW + wid], rbuf_b, sem_b)
        pltpu.make_async_copy(scores_hbm.at[0], rbuf_a, sem_a).wait()
        process(r0 * _NW + wid, rbuf_a, r0 % 8)

        @pl.when(r0 + 2 < _ROWS_PER_W)
        def _issue_a():
            pltpu.async_copy(scores_hbm.at[(r0 + 2) * _NW + wid], rbuf_a,
                             sem_a)

        pltpu.make_async_copy(scores_hbm.at[0], rbuf_b, sem_b).wait()
        process(r1 * _NW + wid, rbuf_b, r1 % 8)

        @pl.when(t % 4 == 3)
        def _flush():
            b = pl.multiple_of(r1 - 7, 8)
            pltpu.sync_copy(obi, idx_hbm.at[wid, pl.ds(b, 8)])
            pltpu.sync_copy(obs, val_hbm.at[wid, pl.ds(b, 8)])
            pltpu.sync_copy(obo, off_hbm.at[wid, pl.ds(b, 8)])
            pltpu.sync_copy(obm, msk_hbm.at[wid, pl.ds(b, 8)])


@functools.cache
def _get_sc_topk():
    return _sc_topk_partial()(_sc_topk_body)


def _sc_topk_partial():
    return functools.partial(
        pl.kernel,
        out_type=(
            jax.ShapeDtypeStruct((_NW, _ROWS_PER_W, K_PAD), jnp.int32),
            jax.ShapeDtypeStruct((_NW, _ROWS_PER_W, K_PAD), jnp.float32),
            jax.ShapeDtypeStruct((_NW, _ROWS_PER_W, K_PAD), jnp.int32),
            jax.ShapeDtypeStruct((_NW, _ROWS_PER_W, K_PAD), jnp.int32),
        ),
        mesh=plsc.VectorSubcoreMesh(
            core_axis_name="c", subcore_axis_name="s",
            num_cores=_NC, num_subcores=_NS),
        compiler_params=pltpu.CompilerParams(needs_layout_passes=False),
        scratch_types=[
            pltpu.VMEM((N,), jnp.float32),
            pltpu.VMEM((N,), jnp.float32),
            pltpu.VMEM((_NCHUNK,), jnp.float32),
            pltpu.VMEM((16,), jnp.float32),
            pltpu.VMEM((8, K_PAD), jnp.int32),
            pltpu.VMEM((8, K_PAD), jnp.float32),
            pltpu.VMEM((8, K_PAD), jnp.int32),
            pltpu.VMEM((8, K_PAD), jnp.int32),
            pltpu.SemaphoreType.DMA,
            pltpu.SemaphoreType.DMA,
        ],
    )


def kernel(span_emb, mention_scores, num_top_antecedents, W_fast, b_fast,
           emb_fast_distance, W_dist, b_dist):
    del num_top_antecedents
    span_p = jnp.pad(span_emb, ((0, 0), (0, KP - D_SPAN)))
    w_p = jnp.pad(W_fast, ((0, DP - D_SPAN), (0, KP - D_SPAN)))
    b_p = jnp.pad(b_fast, (0, DP - D_SPAN)).reshape(1, DP)

    source_p = _t1(span_p, w_p, b_p)

    bs10 = (emb_fast_distance @ W_dist.T + b_dist)[:, 0]
    bs_pad = jnp.zeros((1, 128), jnp.float32).at[0, :10].set(bs10)
    ms_row = mention_scores.reshape(N, 1)
    ms_col = mention_scores.reshape(1, N)

    scores = _t2(source_p, span_p, ms_row, ms_col, bs_pad)

    idx, val, off, msk = _get_sc_topk()(scores)

    def _unshard(x):
        return x.transpose(1, 0, 2).reshape(N, K_PAD)[:, :K_TOP]

    return (_unshard(idx), _unshard(msk).astype(bool),
            _unshard(val), _unshard(off))
